# Initial kernel scaffold; baseline (speedup 1.0000x reference)
import jax
import jax.numpy as jnp
from jax.experimental import pallas as pl


def kernel(users_emb, items_emb, edge_index, edge_weight):
    raise NotImplementedError("write your pallas kernel here")


# speedup vs baseline: 4.0956x; 4.0956x over previous
import functools

import jax
import jax.numpy as jnp
from jax import lax
from jax.experimental import pallas as pl
from jax.experimental.pallas import tpu as pltpu
from jax.experimental.pallas import tpu_sc as plsc

_NC = 2
_NS = 16
_L = 16
_W = _NC * _NS
_C = 80


def _sc_gather_scatter(emb, src, dst, w, zeros):
    N, D = emb.shape
    E = src.shape[0]
    assert E % (_W * _C) == 0 and N % _NS == 0 and D % _L == 0
    epw = E // _W
    nch = epw // _C
    stripe = ((N // _NS + 7) // 8) * 8
    last_stripe = N - stripe * (_NS - 1)
    assert last_stripe > 0 and last_stripe % 8 == 0

    mesh = plsc.VectorSubcoreMesh(core_axis_name="c", subcore_axis_name="s")

    @functools.partial(
        pl.kernel,
        mesh=mesh,
        out_type=jax.ShapeDtypeStruct((_NC, N, D), jnp.float32),
        scratch_types=[
            pltpu.VMEM((_C,), jnp.int32),
            pltpu.VMEM((_C,), jnp.int32),
            pltpu.VMEM((_C,), jnp.float32),
            pltpu.VMEM((_C, D), jnp.float32),
            pltpu.VMEM_SHARED((N, D), jnp.float32),
            pltpu.SemaphoreType.DMA,
        ],
    )
    def k(emb_hbm, src_hbm, dst_hbm, w_hbm, zeros_hbm, out_hbm,
          src_v, dst_v, w_v, rows_v, acc_sh, sem):
        cid = lax.axis_index("c")
        sid = lax.axis_index("s")
        wid = sid * _NC + cid

        r0 = sid * stripe

        @pl.when(sid < _NS - 1)
        def _():
            pltpu.sync_copy(zeros_hbm.at[pl.ds(r0, stripe)],
                            acc_sh.at[pl.ds(r0, stripe)])

        @pl.when(sid == _NS - 1)
        def _():
            pltpu.sync_copy(zeros_hbm.at[pl.ds(r0, last_stripe)],
                            acc_sh.at[pl.ds(r0, last_stripe)])

        plsc.subcore_barrier()

        base0 = wid * epw

        def chunk(c, carry):
            base = pl.multiple_of(base0 + c * _C, 8)
            pltpu.sync_copy(src_hbm.at[pl.ds(base, _C)], src_v)
            pltpu.sync_copy(dst_hbm.at[pl.ds(base, _C)], dst_v)
            pltpu.sync_copy(w_hbm.at[pl.ds(base, _C)], w_v)
            pltpu.async_copy(emb_hbm.at[src_v], rows_v, sem).wait()

            def group(g, cc):
                wv = w_v[pl.ds(g * _L, _L)]
                for e in range(_L):
                    wb = wv[e]
                    r = g * _L + e
                    for j in range(D // _L):
                        sl = pl.ds(j * _L, _L)
                        rows_v[r, sl] = rows_v[r, sl] * wb
                return cc

            lax.fori_loop(0, _C // _L, group, 0)
            pltpu.sync_copy(rows_v, acc_sh.at[dst_v], add=True)
            return carry

        lax.fori_loop(0, nch, chunk, 0)
        plsc.subcore_barrier()

        @pl.when(sid < _NS - 1)
        def _():
            pltpu.sync_copy(acc_sh.at[pl.ds(r0, stripe)],
                            out_hbm.at[cid, pl.ds(r0, stripe)])

        @pl.when(sid == _NS - 1)
        def _():
            pltpu.sync_copy(acc_sh.at[pl.ds(r0, last_stripe)],
                            out_hbm.at[cid, pl.ds(r0, last_stripe)])

    return k(emb, src, dst, w, zeros)


def _combine(partials):
    _, N, D = partials.shape

    def body(p_ref, o_ref):
        o_ref[...] = p_ref[0] + p_ref[1]

    return pl.pallas_call(
        body,
        out_shape=jax.ShapeDtypeStruct((N, D), jnp.float32),
    )(partials)


def kernel(users_emb, items_emb, edge_index, edge_weight):
    num_user = users_emb.shape[0]
    emb = jnp.concatenate([users_emb, items_emb], axis=0)
    src = edge_index[0]
    dst = edge_index[1]
    zeros = jnp.zeros(emb.shape, jnp.float32)
    partials = _sc_gather_scatter(emb, src, dst, edge_weight, zeros)
    out = _combine(partials)
    return out[:num_user], out[num_user:]


# speedup vs baseline: 9.4840x; 2.3156x over previous
import functools

import jax
import jax.numpy as jnp
from jax import lax
from jax.experimental import pallas as pl
from jax.experimental.pallas import tpu as pltpu
from jax.experimental.pallas import tpu_sc as plsc

_NC = 2
_NS = 16
_L = 16
_W = _NC * _NS
_C = 80
_R = 4


def _sc_gather_scatter(emb, src3, dst3, w3, zeros):
    N, D = emb.shape
    _, nch, _ = src3.shape
    assert nch % _R == 1
    nquad = (nch - 1) // _R
    stripe = ((N // _NS + 7) // 8) * 8
    last_stripe = N - stripe * (_NS - 1)
    assert last_stripe > 0 and last_stripe % 8 == 0

    mesh = plsc.VectorSubcoreMesh(core_axis_name="c", subcore_axis_name="s")

    @functools.partial(
        pl.kernel,
        mesh=mesh,
        out_type=jax.ShapeDtypeStruct((_NC, N, D), jnp.float32),
        scratch_types=[
            pltpu.VMEM((_R, _C), jnp.int32),
            pltpu.VMEM((_R, _C), jnp.int32),
            pltpu.VMEM((_R, _C), jnp.float32),
            pltpu.VMEM((_C, D), jnp.float32),
            pltpu.VMEM((_C, D), jnp.float32),
            pltpu.VMEM_SHARED((N, D), jnp.float32),
            pltpu.SemaphoreType.DMA,
            pltpu.SemaphoreType.DMA,
            [pltpu.SemaphoreType.DMA] * _R,
        ],
    )
    def k(emb_hbm, src_hbm, dst_hbm, w_hbm, zeros_hbm, out_hbm,
          src_v, dst_v, w_v, rows0, rows1, acc_sh, g0, g1, esems):
        cid = lax.axis_index("c")
        sid = lax.axis_index("s")
        wid = sid * _NC + cid
        rows = (rows0, rows1)
        gsems = (g0, g1)

        r0 = sid * stripe

        @pl.when(sid < _NS - 1)
        def _():
            pltpu.sync_copy(zeros_hbm.at[pl.ds(r0, stripe)],
                            acc_sh.at[pl.ds(r0, stripe)])

        @pl.when(sid == _NS - 1)
        def _():
            pltpu.sync_copy(zeros_hbm.at[pl.ds(r0, last_stripe)],
                            acc_sh.at[pl.ds(r0, last_stripe)])

        plsc.subcore_barrier()

        def edges(c, slot):
            return (
                pltpu.make_async_copy(src_hbm.at[wid, c], src_v.at[slot],
                                      esems[slot]),
                pltpu.make_async_copy(dst_hbm.at[wid, c], dst_v.at[slot],
                                      esems[slot]),
                pltpu.make_async_copy(w_hbm.at[wid, c], w_v.at[slot],
                                      esems[slot]),
            )

        def gather(slot, rb):
            return pltpu.make_async_copy(emb_hbm.at[src_v.at[slot]],
                                         rows[rb], gsems[rb])

        def scale(slot, rb):
            rv = rows[rb]

            def group(g, cc):
                wv = w_v[slot, pl.ds(g * _L, _L)]
                for e in range(_L):
                    wb = wv[e]
                    r = g * _L + e
                    for j in range(D // _L):
                        sl = pl.ds(j * _L, _L)
                        rv[r, sl] = rv[r, sl] * wb
                return cc

            lax.fori_loop(0, _C // _L, group, 0)

        def scatter(slot, rb):
            pltpu.sync_copy(rows[rb], acc_sh.at[dst_v.at[slot]], add=True)

        for s in range(_R - 1):
            for d in edges(s, s):
                d.start()
        for d in edges(0, 0):
            d.wait()
        gather(0, 0).start()

        def quad(q, carry):
            c0 = q * _R
            for k in range(_R):
                c = c0 + k
                rb = k & 1
                nslot = (k + 1) % _R
                for d in edges(0, nslot):
                    d.wait()
                gather(nslot, 1 - rb).start()
                gather(k, rb).wait()
                scale(k, rb)
                scatter(k, rb)

                @pl.when(c + _R - 1 < nch)
                def _():
                    for d in edges(c + _R - 1, (k + _R - 1) % _R):
                        d.start()

            return carry

        lax.fori_loop(0, nquad, quad, 0)
        lslot = (nch - 1) % _R
        gather(lslot, 0).wait()
        scale(lslot, 0)
        scatter(lslot, 0)

        plsc.subcore_barrier()

        @pl.when(sid < _NS - 1)
        def _():
            pltpu.sync_copy(acc_sh.at[pl.ds(r0, stripe)],
                            out_hbm.at[cid, pl.ds(r0, stripe)])

        @pl.when(sid == _NS - 1)
        def _():
            pltpu.sync_copy(acc_sh.at[pl.ds(r0, last_stripe)],
                            out_hbm.at[cid, pl.ds(r0, last_stripe)])

    return k(emb, src3, dst3, w3, zeros)


def _combine(partials):
    _, N, D = partials.shape

    def body(p_ref, o_ref):
        o_ref[...] = p_ref[0] + p_ref[1]

    return pl.pallas_call(
        body,
        out_shape=jax.ShapeDtypeStruct((N, D), jnp.float32),
    )(partials)


def kernel(users_emb, items_emb, edge_index, edge_weight):
    num_user = users_emb.shape[0]
    emb = jnp.concatenate([users_emb, items_emb], axis=0)
    E = edge_weight.shape[0]
    epw = E // _W
    nch = epw // _C
    assert epw % _C == 0
    src3 = edge_index[0].reshape(_W, nch, _C)
    dst3 = edge_index[1].reshape(_W, nch, _C)
    w3 = edge_weight.reshape(_W, nch, _C)
    zeros = jnp.zeros(emb.shape, jnp.float32)
    partials = _sc_gather_scatter(emb, src3, dst3, w3, zeros)
    out = _combine(partials)
    return out[:num_user], out[num_user:]


# speedup vs baseline: 10.0617x; 1.0609x over previous
import functools

import jax
import jax.numpy as jnp
from jax import lax
from jax.experimental import pallas as pl
from jax.experimental.pallas import tpu as pltpu
from jax.experimental.pallas import tpu_sc as plsc

_NC = 2
_NS = 16
_L = 16
_W = _NC * _NS
_C = 80
_R = 4


def _sc_gather_scatter(emb, src3, dst3, w3):
    N, D = emb.shape
    _, nch, _ = src3.shape
    assert nch % _R == 1
    nquad = (nch - 1) // _R
    stripe = ((N // _NS + 7) // 8) * 8
    last_stripe = N - stripe * (_NS - 1)
    assert last_stripe > 0 and last_stripe % 8 == 0
    assert stripe % 8 == 0 and _C % 8 == 0

    mesh = plsc.VectorSubcoreMesh(core_axis_name="c", subcore_axis_name="s")

    @functools.partial(
        pl.kernel,
        mesh=mesh,
        out_type=jax.ShapeDtypeStruct((_NC, N, D), jnp.float32),
        scratch_types=[
            pltpu.VMEM((_R, _C), jnp.int32),
            pltpu.VMEM((_R, _C), jnp.int32),
            pltpu.VMEM((_R, _C), jnp.float32),
            pltpu.VMEM((_C, D), jnp.float32),
            pltpu.VMEM((_C, D), jnp.float32),
            pltpu.VMEM_SHARED((N, D), jnp.float32),
            pltpu.SemaphoreType.DMA,
            pltpu.SemaphoreType.DMA,
            pltpu.SemaphoreType.DMA,
            pltpu.SemaphoreType.DMA,
            [pltpu.SemaphoreType.DMA] * _R,
        ],
    )
    def k(emb_hbm, src_hbm, dst_hbm, w_hbm, out_hbm,
          src_v, dst_v, w_v, rows0, rows1, acc_sh, g0, g1, s0, s1, esems):
        cid = lax.axis_index("c")
        sid = lax.axis_index("s")
        wid = sid * _NC + cid
        rows = (rows0, rows1)
        gsems = (g0, g1)
        ssems = (s0, s1)

        r0 = sid * stripe
        zv = jnp.zeros((_L,), jnp.float32)

        def zrow(r, cc):
            for j in range(D // _L):
                rows0[r, pl.ds(j * _L, _L)] = zv
            return cc

        lax.fori_loop(0, _C, zrow, 0)

        def zero_stripe(span):
            nfull, rem = span // _C, span % _C
            assert rem % 8 == 0
            for i in range(nfull):
                pltpu.sync_copy(rows0,
                                acc_sh.at[pl.ds(r0 + i * _C, _C)])
            if rem:
                pltpu.sync_copy(rows0.at[pl.ds(0, rem)],
                                acc_sh.at[pl.ds(r0 + nfull * _C, rem)])

        @pl.when(sid < _NS - 1)
        def _():
            zero_stripe(stripe)

        @pl.when(sid == _NS - 1)
        def _():
            zero_stripe(last_stripe)

        plsc.subcore_barrier()

        def edges(c, slot):
            return (
                pltpu.make_async_copy(src_hbm.at[wid, c], src_v.at[slot],
                                      esems[slot]),
                pltpu.make_async_copy(dst_hbm.at[wid, c], dst_v.at[slot],
                                      esems[slot]),
                pltpu.make_async_copy(w_hbm.at[wid, c], w_v.at[slot],
                                      esems[slot]),
            )

        def gather(slot, rb):
            return pltpu.make_async_copy(emb_hbm.at[src_v.at[slot]],
                                         rows[rb], gsems[rb])

        def scale(slot, rb):
            rv = rows[rb]

            def group(g, cc):
                wv = w_v[slot, pl.ds(g * _L, _L)]
                for e in range(_L):
                    wb = wv[e]
                    r = g * _L + e
                    for j in range(D // _L):
                        sl = pl.ds(j * _L, _L)
                        rv[r, sl] = rv[r, sl] * wb
                return cc

            lax.fori_loop(0, _C // _L, group, 0)

        def scatter(slot, rb):
            return pltpu.make_async_copy(rows[rb], acc_sh.at[dst_v.at[slot]],
                                         ssems[rb])

        for s in range(_R - 1):
            for d in edges(s, s):
                d.start()
        for d in edges(0, 0):
            d.wait()
        gather(0, 0).start()

        def quad(q, carry):
            c0 = q * _R
            for k in range(_R):
                c = c0 + k
                rb = k & 1
                nslot = (k + 1) % _R
                for d in edges(0, nslot):
                    d.wait()

                if k == 0:
                    @pl.when(q > 0)
                    def _():
                        scatter(_R - 1, 1 - rb).wait()
                else:
                    scatter(k - 1, 1 - rb).wait()

                gather(nslot, 1 - rb).start()
                gather(k, rb).wait()
                scale(k, rb)
                scatter(k, rb).start(add=True)

                @pl.when(c + _R - 1 < nch)
                def _():
                    for d in edges(c + _R - 1, (k + _R - 1) % _R):
                        d.start()

            return carry

        lax.fori_loop(0, nquad, quad, 0)
        lslot = (nch - 1) % _R
        scatter((lslot + _R - 1) % _R, 1).wait()
        gather(lslot, 0).wait()
        scale(lslot, 0)
        scatter(lslot, 0).start(add=True)
        scatter(lslot, 0).wait()

        plsc.subcore_barrier()

        @pl.when(sid < _NS - 1)
        def _():
            pltpu.sync_copy(acc_sh.at[pl.ds(r0, stripe)],
                            out_hbm.at[cid, pl.ds(r0, stripe)])

        @pl.when(sid == _NS - 1)
        def _():
            pltpu.sync_copy(acc_sh.at[pl.ds(r0, last_stripe)],
                            out_hbm.at[cid, pl.ds(r0, last_stripe)])

    return k(emb, src3, dst3, w3)


def _combine_split(partials, num_user):
    _, N, D = partials.shape
    num_item = N - num_user

    def body(p_ref, u_ref, i_ref):
        u_ref[...] = p_ref[0, :num_user] + p_ref[1, :num_user]
        i_ref[...] = p_ref[0, num_user:] + p_ref[1, num_user:]

    return pl.pallas_call(
        body,
        out_shape=(jax.ShapeDtypeStruct((num_user, D), jnp.float32),
                   jax.ShapeDtypeStruct((num_item, D), jnp.float32)),
    )(partials)


def kernel(users_emb, items_emb, edge_index, edge_weight):
    num_user = users_emb.shape[0]
    emb = jnp.concatenate([users_emb, items_emb], axis=0)
    E = edge_weight.shape[0]
    epw = E // _W
    nch = epw // _C
    assert epw % _C == 0
    src3 = edge_index[0].reshape(_W, nch, _C)
    dst3 = edge_index[1].reshape(_W, nch, _C)
    w3 = edge_weight.reshape(_W, nch, _C)
    partials = _sc_gather_scatter(emb, src3, dst3, w3)
    return _combine_split(partials, num_user)
